# baseline (speedup 1.0000x reference)
import functools

import jax
import jax.numpy as jnp
from jax import lax
from jax.experimental import pallas as pl
from jax.experimental.pallas import tpu as pltpu
from jax.experimental.pallas import tpu_sc as plsc

B = 32
S = 8
MAX_K = 32
V = 100000
TEMPERATURE = 10.0

R = B * S
N = R * V
NC = 2
NS = 16
NW = NC * NS
ROWS_PER_W = R // NW
L = 16
E = ROWS_PER_W * MAX_K


def _memset_body(o_ref):
    o_ref[...] = jnp.zeros_like(o_ref)


_memset = pl.pallas_call(
    _memset_body,
    out_shape=jax.ShapeDtypeStruct((R, V), jnp.float32),
    grid=(32,),
    out_specs=pl.BlockSpec((R // 32, V), lambda i: (i, 0)),
)


def _scatter_body(vals_hbm, dist_hbm, buf_hbm, vals_v, dist_v, ibuf, vbuf, sem):
    wid = lax.axis_index("s") * NC + lax.axis_index("c")
    base = wid * ROWS_PER_W

    pltpu.sync_copy(vals_hbm.at[pl.ds(base * MAX_K, E)], vals_v)
    pltpu.sync_copy(dist_hbm.at[pl.ds(base * MAX_K, E)], dist_v)

    lane_iota = lax.iota(jnp.int32, L)

    for r in range(ROWS_PER_W):
        v0 = vals_v[pl.ds(r * MAX_K, L)]
        v1 = vals_v[pl.ds(r * MAX_K + L, L)]
        d0 = dist_v[pl.ds(r * MAX_K, L)]
        d1 = dist_v[pl.ds(r * MAX_K + L, L)]

        e0 = jnp.exp(d0 * (-1.0 / TEMPERATURE))
        e1 = jnp.exp(d1 * (-1.0 / TEMPERATURE))
        t = e0 + e1
        for sh in (8, 4, 2, 1):
            t = t + t.at[lane_iota ^ sh].get(mode="promise_in_bounds")
        inv = 1.0 / t
        w0 = e0 * inv
        w1 = e1 * inv

        t0 = jnp.zeros((L,), jnp.float32)
        t1 = jnp.zeros((L,), jnp.float32)
        for vv, ww in ((v0, w0), (v1, w1)):
            for j in range(L):
                sj = jnp.full((L,), j, jnp.int32)
                bv = vv.at[sj].get(mode="promise_in_bounds")
                bw = ww.at[sj].get(mode="promise_in_bounds")
                t0 = t0 + jnp.where(v0 == bv, bw, 0.0)
                t1 = t1 + jnp.where(v1 == bv, bw, 0.0)

        g0 = (base + r) * V + v0
        g1 = (base + r) * V + v1

        row = r // 4
        off = (r % 4) * MAX_K
        ibuf[row, pl.ds(off, L)] = g0
        ibuf[row, pl.ds(off + L, L)] = g1
        vbuf[row, pl.ds(off, L)] = t0
        vbuf[row, pl.ds(off + L, L)] = t1

    for j in range(2):
        pltpu.async_copy(vbuf.at[j], buf_hbm.at[ibuf.at[j]], sem).wait()


_scatter = functools.partial(
    pl.kernel,
    mesh=plsc.VectorSubcoreMesh(core_axis_name="c", subcore_axis_name="s"),
    scratch_types=[
        pltpu.VMEM((E,), jnp.int32),
        pltpu.VMEM((E,), jnp.float32),
        pltpu.VMEM((2, 128), jnp.int32),
        pltpu.VMEM((2, 128), jnp.float32),
        pltpu.SemaphoreType.DMA,
    ],
    compiler_params=pltpu.CompilerParams(needs_layout_passes=False),
)(_scatter_body)


def kernel(vals, distances):
    vals_flat = vals.reshape(R * MAX_K).astype(jnp.int32)
    dist_flat = distances.reshape(R * MAX_K).astype(jnp.float32)
    buf = jax.new_ref(_memset().reshape(N))
    _scatter(vals_flat, dist_flat, buf)
    return buf[...].reshape(B, S, V)


# speedup vs baseline: 1.6956x; 1.6956x over previous
import functools

import jax
import jax.numpy as jnp
from jax import lax
from jax.experimental import pallas as pl
from jax.experimental.pallas import tpu as pltpu
from jax.experimental.pallas import tpu_sc as plsc

B = 32
S = 8
MAX_K = 32
V = 100000
TEMPERATURE = 10.0

R = B * S
N = R * V
NC = 2
NS = 16
NW = NC * NS
ROWS_PER_W = R // NW
L = 16
E = ROWS_PER_W * MAX_K


def _memset_body(o_ref):
    o_ref[...] = jnp.zeros_like(o_ref)


_memset = pl.pallas_call(
    _memset_body,
    out_shape=jax.ShapeDtypeStruct((N // 128, 128), jnp.float32),
    grid=(25,),
    out_specs=pl.BlockSpec((N // 128 // 25, 128), lambda i: (i, 0)),
)


def _scatter_body(vals_hbm, dist_hbm, buf_hbm, vals_v, dist_v, ibuf, vbuf, sem):
    wid = lax.axis_index("s") * NC + lax.axis_index("c")
    base = wid * ROWS_PER_W

    pltpu.sync_copy(vals_hbm.at[pl.ds(base * MAX_K, E)], vals_v)
    pltpu.sync_copy(dist_hbm.at[pl.ds(base * MAX_K, E)], dist_v)

    lane_iota = lax.iota(jnp.int32, L)

    for r in range(ROWS_PER_W):
        v0 = vals_v[pl.ds(r * MAX_K, L)]
        v1 = vals_v[pl.ds(r * MAX_K + L, L)]
        d0 = dist_v[pl.ds(r * MAX_K, L)]
        d1 = dist_v[pl.ds(r * MAX_K + L, L)]

        e0 = jnp.exp(d0 * (-1.0 / TEMPERATURE))
        e1 = jnp.exp(d1 * (-1.0 / TEMPERATURE))
        t = e0 + e1
        for sh in (8, 4, 2, 1):
            t = t + t.at[lane_iota ^ sh].get(mode="promise_in_bounds")
        inv = 1.0 / t
        w0 = e0 * inv
        w1 = e1 * inv

        t0 = jnp.zeros((L,), jnp.float32)
        t1 = jnp.zeros((L,), jnp.float32)
        for vv, ww in ((v0, w0), (v1, w1)):
            for j in range(L):
                sj = jnp.full((L,), j, jnp.int32)
                bv = vv.at[sj].get(mode="promise_in_bounds")
                bw = ww.at[sj].get(mode="promise_in_bounds")
                t0 = t0 + jnp.where(v0 == bv, bw, 0.0)
                t1 = t1 + jnp.where(v1 == bv, bw, 0.0)

        g0 = (base + r) * V + v0
        g1 = (base + r) * V + v1

        row = r // 4
        off = (r % 4) * MAX_K
        ibuf[row, pl.ds(off, L)] = g0
        ibuf[row, pl.ds(off + L, L)] = g1
        vbuf[row, pl.ds(off, L)] = t0
        vbuf[row, pl.ds(off + L, L)] = t1

    for j in range(2):
        pltpu.async_copy(vbuf.at[j], buf_hbm.at[ibuf.at[j]], sem).wait()


_scatter = functools.partial(
    pl.kernel,
    mesh=plsc.VectorSubcoreMesh(core_axis_name="c", subcore_axis_name="s"),
    scratch_types=[
        pltpu.VMEM((E,), jnp.int32),
        pltpu.VMEM((E,), jnp.float32),
        pltpu.VMEM((2, 128), jnp.int32),
        pltpu.VMEM((2, 128), jnp.float32),
        pltpu.SemaphoreType.DMA,
    ],
    compiler_params=pltpu.CompilerParams(needs_layout_passes=False),
)(_scatter_body)


def kernel(vals, distances):
    vals_flat = vals.reshape(R * MAX_K).astype(jnp.int32)
    dist_flat = distances.reshape(R * MAX_K).astype(jnp.float32)
    buf = jax.new_ref(_memset().reshape(N))
    _scatter(vals_flat, dist_flat, buf)
    return buf[...].reshape(B, S, V)


# speedup vs baseline: 5.7432x; 3.3872x over previous
import functools

import jax
import jax.numpy as jnp
from jax import lax
from jax.experimental import pallas as pl
from jax.experimental.pallas import tpu as pltpu
from jax.experimental.pallas import tpu_sc as plsc

B = 32
S = 8
MAX_K = 32
V = 100000
TEMPERATURE = 10.0

R = B * S
NC = 2
NS = 16
NW = NC * NS
ROWS_PER_W = R // NW
L = 16
E = ROWS_PER_W * MAX_K


def _memset_body(o_ref):
    o_ref[...] = jnp.zeros_like(o_ref)


_memset = pl.pallas_call(
    _memset_body,
    out_shape=jax.ShapeDtypeStruct((R, V), jnp.float32),
    grid=(32,),
    out_specs=pl.BlockSpec((R // 32, V), lambda i: (i, 0)),
)


def _scatter_body(vals_hbm, dist_hbm, buf_hbm, vals_v, dist_v, slots, sem):
    wid = lax.axis_index("s") * NC + lax.axis_index("c")
    base = wid * ROWS_PER_W

    pltpu.sync_copy(vals_hbm.at[pl.ds(base * MAX_K, E)], vals_v)
    pltpu.sync_copy(dist_hbm.at[pl.ds(base * MAX_K, E)], dist_v)

    lane_iota = lax.iota(jnp.int32, L)
    zeros16 = jnp.zeros((L,), jnp.float32)

    for r in range(ROWS_PER_W):
        v0 = vals_v[pl.ds(r * MAX_K, L)]
        v1 = vals_v[pl.ds(r * MAX_K + L, L)]
        d0 = dist_v[pl.ds(r * MAX_K, L)]
        d1 = dist_v[pl.ds(r * MAX_K + L, L)]

        e0 = jnp.exp(d0 * (-1.0 / TEMPERATURE))
        e1 = jnp.exp(d1 * (-1.0 / TEMPERATURE))
        t = e0 + e1
        for sh in (8, 4, 2, 1):
            t = t + t.at[lane_iota ^ sh].get(mode="promise_in_bounds")
        inv = 1.0 / t
        w0 = e0 * inv
        w1 = e1 * inv

        def mk_step(vsrc, wsrc, off):
            def step(j, carry):
                t0, t1, a0, a1 = carry
                jv = jnp.zeros((L,), jnp.int32) + j
                bv = vsrc.at[jv].get(mode="promise_in_bounds")
                bw = wsrc.at[jv].get(mode="promise_in_bounds")
                m0 = v0 == bv
                m1 = v1 == bv
                gj = j + off
                t0 = t0 + jnp.where(m0, bw, 0.0)
                t1 = t1 + jnp.where(m1, bw, 0.0)
                a0 = a0 + jnp.where(m0 & (gj < lane_iota), 1, 0)
                a1 = a1 + jnp.where(m1 & (gj < lane_iota + L), 1, 0)
                return t0, t1, a0, a1

            return step

        zi = jnp.zeros((L,), jnp.int32)
        t0, t1, a0, a1 = lax.fori_loop(
            0, L, mk_step(v0, w0, 0), (zeros16, zeros16, zi, zi)
        )
        t0, t1, a0, a1 = lax.fori_loop(
            0, L, mk_step(v1, w1, L), (t0, t1, a0, a1)
        )
        keep0 = a0 == 0
        keep1 = a1 == 0

        def fill_slot(slot_idx, sb, width):
            off0 = v0 - sb
            off1 = v1 - sb
            in0 = (off0 >= 0) & (off0 < width) & keep0
            in1 = (off1 >= 0) & (off1 < width) & keep1
            idx0 = jnp.where(in0, off0, 127)
            idx1 = jnp.where(in1, off1, 127)
            kv = jnp.zeros((L,), jnp.int32) + slot_idx
            for q in range(8):
                slots[slot_idx, pl.ds(q * L, L)] = zeros16
            plsc.addupdate_scatter(slots, [kv, idx0], t0, mask=in0)
            plsc.addupdate_scatter(slots, [kv, idx1], t1, mask=in1)

        copies = []
        for k in range(MAX_K):
            vk = v0 if k < L else v1
            c = jnp.sum(jnp.where(lane_iota == (k % L), vk, 0))
            sb = pl.multiple_of(
                jnp.minimum(jnp.bitwise_and(c, -128), V - 160), 128
            )
            fill_slot(k, sb, 128)
            copies.append(
                pltpu.async_copy(
                    slots.at[k, pl.ds(0, 128)],
                    buf_hbm.at[base + r].at[pl.ds(sb, 128)],
                    sem,
                )
            )
        tail = V - (V % 128)
        fill_slot(MAX_K, jnp.int32(tail), V % 128)
        copies.append(
            pltpu.async_copy(
                slots.at[MAX_K, pl.ds(0, V % 128)],
                buf_hbm.at[base + r].at[pl.ds(tail, V % 128)],
                sem,
            )
        )
        for cp in copies:
            cp.wait()


_scatter = functools.partial(
    pl.kernel,
    mesh=plsc.VectorSubcoreMesh(core_axis_name="c", subcore_axis_name="s"),
    scratch_types=[
        pltpu.VMEM((E,), jnp.int32),
        pltpu.VMEM((E,), jnp.float32),
        pltpu.VMEM((MAX_K + 1, 128), jnp.float32),
        pltpu.SemaphoreType.DMA,
    ],
    compiler_params=pltpu.CompilerParams(needs_layout_passes=False),
)(_scatter_body)


def kernel(vals, distances):
    vals_flat = vals.reshape(R * MAX_K).astype(jnp.int32)
    dist_flat = distances.reshape(R * MAX_K).astype(jnp.float32)
    buf = jax.new_ref(_memset())
    _scatter(vals_flat, dist_flat, buf)
    return buf[...].reshape(B, S, V)


# speedup vs baseline: 5.8352x; 1.0160x over previous
import functools

import jax
import jax.numpy as jnp
from jax import lax
from jax.experimental import pallas as pl
from jax.experimental.pallas import tpu as pltpu
from jax.experimental.pallas import tpu_sc as plsc

B = 32
S = 8
MAX_K = 32
V = 100000
TEMPERATURE = 10.0

R = B * S
NC = 2
NS = 16
NW = NC * NS
ROWS_PER_W = R // NW
L = 16
E = ROWS_PER_W * MAX_K


def _memset_body(o_ref):
    o_ref[...] = jnp.zeros_like(o_ref)


_memset = pl.pallas_call(
    _memset_body,
    out_shape=jax.ShapeDtypeStruct((R, V), jnp.float32),
    grid=(32,),
    out_specs=pl.BlockSpec((R // 32, V), lambda i: (i, 0)),
)


def _scatter_body(vals_hbm, dist_hbm, buf_hbm, vals_v, dist_v, slots, sem):
    wid = lax.axis_index("s") * NC + lax.axis_index("c")
    base = wid * ROWS_PER_W

    pltpu.sync_copy(vals_hbm.at[pl.ds(base * MAX_K, E)], vals_v)
    pltpu.sync_copy(dist_hbm.at[pl.ds(base * MAX_K, E)], dist_v)

    lane_iota = lax.iota(jnp.int32, L)
    zeros16 = jnp.zeros((L,), jnp.float32)
    copies = []

    for r in range(ROWS_PER_W):
        v0 = vals_v[pl.ds(r * MAX_K, L)]
        v1 = vals_v[pl.ds(r * MAX_K + L, L)]
        d0 = dist_v[pl.ds(r * MAX_K, L)]
        d1 = dist_v[pl.ds(r * MAX_K + L, L)]

        e0 = jnp.exp(d0 * (-1.0 / TEMPERATURE))
        e1 = jnp.exp(d1 * (-1.0 / TEMPERATURE))
        t = e0 + e1
        for sh in (8, 4, 2, 1):
            t = t + t.at[lane_iota ^ sh].get(mode="promise_in_bounds")
        inv = 1.0 / t
        w0 = e0 * inv
        w1 = e1 * inv

        def mk_step(vsrc, wsrc, off):
            def step(j, carry):
                t0, t1, a0, a1 = carry
                jv = jnp.zeros((L,), jnp.int32) + j
                bv = vsrc.at[jv].get(mode="promise_in_bounds")
                bw = wsrc.at[jv].get(mode="promise_in_bounds")
                m0 = v0 == bv
                m1 = v1 == bv
                gj = j + off
                t0 = t0 + jnp.where(m0, bw, 0.0)
                t1 = t1 + jnp.where(m1, bw, 0.0)
                a0 = a0 + jnp.where(m0 & (gj < lane_iota), 1, 0)
                a1 = a1 + jnp.where(m1 & (gj < lane_iota + L), 1, 0)
                return t0, t1, a0, a1

            return step

        zi = jnp.zeros((L,), jnp.int32)
        t0, t1, a0, a1 = lax.fori_loop(
            0, L, mk_step(v0, w0, 0), (zeros16, zeros16, zi, zi)
        )
        t0, t1, a0, a1 = lax.fori_loop(
            0, L, mk_step(v1, w1, L), (t0, t1, a0, a1)
        )
        keep0 = a0 == 0
        keep1 = a1 == 0

        def fill_slot(slot_idx0, sb, width):
            slot_idx = r * (MAX_K + 1) + slot_idx0
            off0 = v0 - sb
            off1 = v1 - sb
            in0 = (off0 >= 0) & (off0 < width) & keep0
            in1 = (off1 >= 0) & (off1 < width) & keep1
            idx0 = jnp.where(in0, off0, 127)
            idx1 = jnp.where(in1, off1, 127)
            kv = jnp.zeros((L,), jnp.int32) + slot_idx
            for q in range(8):
                slots[slot_idx, pl.ds(q * L, L)] = zeros16
            plsc.addupdate_scatter(slots, [kv, idx0], t0, mask=in0)
            plsc.addupdate_scatter(slots, [kv, idx1], t1, mask=in1)

        for k in range(MAX_K):
            vk = v0 if k < L else v1
            c = jnp.sum(jnp.where(lane_iota == (k % L), vk, 0))
            sb = pl.multiple_of(
                jnp.minimum(jnp.bitwise_and(c, -128), V - 160), 128
            )
            fill_slot(k, sb, 128)
            copies.append(
                pltpu.async_copy(
                    slots.at[r * (MAX_K + 1) + k, pl.ds(0, 128)],
                    buf_hbm.at[base + r].at[pl.ds(sb, 128)],
                    sem,
                )
            )
        tail = V - (V % 128)
        fill_slot(MAX_K, jnp.int32(tail), V % 128)
        copies.append(
            pltpu.async_copy(
                slots.at[r * (MAX_K + 1) + MAX_K, pl.ds(0, V % 128)],
                buf_hbm.at[base + r].at[pl.ds(tail, V % 128)],
                sem,
            )
        )

    for cp in copies:
        cp.wait()


_scatter = functools.partial(
    pl.kernel,
    mesh=plsc.VectorSubcoreMesh(core_axis_name="c", subcore_axis_name="s"),
    scratch_types=[
        pltpu.VMEM((E,), jnp.int32),
        pltpu.VMEM((E,), jnp.float32),
        pltpu.VMEM((ROWS_PER_W * (MAX_K + 1), 128), jnp.float32),
        pltpu.SemaphoreType.DMA,
    ],
    compiler_params=pltpu.CompilerParams(needs_layout_passes=False),
)(_scatter_body)


def kernel(vals, distances):
    vals_flat = vals.reshape(R * MAX_K).astype(jnp.int32)
    dist_flat = distances.reshape(R * MAX_K).astype(jnp.float32)
    buf = jax.new_ref(_memset())
    _scatter(vals_flat, dist_flat, buf)
    return buf[...].reshape(B, S, V)
